# baseline (speedup 1.0000x reference)
import functools

import jax
import jax.numpy as jnp
from jax import lax
from jax.experimental import pallas as pl
from jax.experimental.pallas import tpu as pltpu
from jax.experimental.pallas import tpu_sc as plsc

_EPC = 64
_NBUF = 3
_CPW = (45, 53)


@functools.lru_cache(maxsize=None)
def _build(e, d):
    info = plsc.get_sparse_core_info()
    nc = info.num_cores
    ns = info.num_subcores
    nw = nc * ns
    n_full = e // _EPC
    tail = e - n_full * _EPC
    n_chunks = n_full + (1 if tail else 0)
    cpw0, cpw1 = _CPW
    if nc != 2 or ns * (cpw0 + cpw1) < n_chunks:
        cpw0 = cpw1 = -(-n_chunks // nw)
    cpw_max = max(cpw0, cpw1)
    epw = cpw_max * _EPC
    last_base_e = (ns * cpw0 + (ns - 1) * cpw1) * _EPC
    last_sz = e - last_base_e
    zfill = (epw - last_sz) // 16
    rounds = -(-cpw_max // _NBUF) + 1

    mesh = plsc.VectorSubcoreMesh(core_axis_name="c", subcore_axis_name="s")

    @functools.partial(
        pl.kernel,
        mesh=mesh,
        out_type=(
            jax.ShapeDtypeStruct((e, 2 * d), jnp.float32),
            jax.ShapeDtypeStruct((e, 2 * d), jnp.float32),
        ),
        scratch_types=(
            [pltpu.VMEM((epw,), jnp.int32) for _ in range(4)]
            + [pltpu.VMEM((_EPC, 2 * d), jnp.float32) for _ in range(2 * _NBUF)]
            + [pltpu.SemaphoreType.DMA for _ in range(4 * _NBUF)]
        ),
    )
    def gather_kernel(data_hbm, eip_hbm, ein_hbm, outp_hbm, outn_hbm, *sc):
        ed = [[sc[2 * s + h] for h in range(2)] for s in range(2)]
        rows = [[sc[4 + s * _NBUF + b] for b in range(_NBUF)] for s in range(2)]
        o = 4 + 2 * _NBUF
        gsem = [[sc[o + s * _NBUF + b] for b in range(_NBUF)] for s in range(2)]
        o += 2 * _NBUF
        wsem = [[sc[o + s * _NBUF + b] for b in range(_NBUF)] for s in range(2)]
        outs = (outp_hbm, outn_hbm)
        eis = (eip_hbm, ein_hbm)

        c_idx = lax.axis_index("c")
        s_idx = lax.axis_index("s")
        base_chunk = jnp.where(c_idx == 0, s_idx * cpw0,
                               ns * cpw0 + s_idx * cpw1)
        my_cpw = jnp.where(c_idx == 0, cpw0, cpw1)
        base_e = base_chunk * _EPC
        is_last = jnp.logical_and(c_idx == nc - 1, s_idx == ns - 1)

        @pl.when(jnp.logical_not(is_last))
        def _stage_full():
            for s in range(2):
                for h in range(2):
                    pltpu.async_copy(eis[s].at[pl.ds(h * e + base_e, epw)],
                                     ed[s][h], gsem[s][h])
            for s in range(2):
                for h in range(2):
                    pltpu.make_async_copy(eis[s].at[pl.ds(h * e + base_e, epw)],
                                          ed[s][h], gsem[s][h]).wait()

        @pl.when(is_last)
        def _stage_last():
            for s in range(2):
                pltpu.async_copy(eis[s].at[pl.ds(base_e, epw)],
                                 ed[s][0], gsem[s][0])
                pltpu.async_copy(eis[s].at[pl.ds(e + base_e, last_sz)],
                                 ed[s][1].at[pl.ds(0, last_sz)], gsem[s][1])
            for s in range(2):
                pltpu.make_async_copy(eis[s].at[pl.ds(base_e, epw)],
                                      ed[s][0], gsem[s][0]).wait()
                pltpu.make_async_copy(eis[s].at[pl.ds(e + base_e, last_sz)],
                                      ed[s][1].at[pl.ds(0, last_sz)],
                                      gsem[s][1]).wait()
            zeros = jnp.zeros((16,), jnp.int32)
            for s in range(2):
                for g in range(zfill):
                    ed[s][1][pl.ds(last_sz + g * 16, 16)] = zeros

        def write_desc(s, b, cg, wait):
            @pl.when(cg < n_full)
            def _full():
                cp = pltpu.make_async_copy(
                    rows[s][b], outs[s].at[pl.ds(cg * _EPC, _EPC)], wsem[s][b])
                cp.wait() if wait else cp.start()
            if tail:
                @pl.when(cg == n_full)
                def _part():
                    cp = pltpu.make_async_copy(
                        rows[s][b].at[pl.ds(0, tail)],
                        outs[s].at[pl.ds(n_full * _EPC, tail)], wsem[s][b])
                    cp.wait() if wait else cp.start()

        def gather_desc(s, b, c_rel, wait):
            for h in range(2):
                cp = pltpu.make_async_copy(
                    data_hbm.at[ed[s][h].at[pl.ds(c_rel * _EPC, _EPC)]],
                    rows[s][b].at[:, pl.ds(h * d, d)], gsem[s][b])
                cp.wait() if wait else cp.start()

        def round_(j, carry):
            for b in range(_NBUF):
                c_new = j * _NBUF + b
                c_mid = c_new - _NBUF

                @pl.when(jnp.logical_and(c_mid >= 0, c_mid < my_cpw))
                def _drain_and_write():
                    for s in (0, 1):
                        gather_desc(s, b, c_mid, wait=True)
                    for s in (0, 1):
                        write_desc(s, b, base_chunk + c_mid, wait=False)

                @pl.when(c_new < my_cpw)
                def _fire():
                    @pl.when(c_mid >= 0)
                    def _wait_prev_write():
                        for s in (0, 1):
                            write_desc(s, b, base_chunk + c_mid, wait=True)
                    for s in (0, 1):
                        gather_desc(s, b, c_new, wait=False)

            return carry

        lax.fori_loop(0, rounds, round_, 0)

        for b in range(_NBUF):
            last_c = ((my_cpw - 1 - b) // _NBUF) * _NBUF + b

            @pl.when(last_c >= 0)
            def _drain_final():
                for s in (0, 1):
                    write_desc(s, b, base_chunk + last_c, wait=True)

    return gather_kernel


def kernel(data, edge_index_pos, edge_index_neg):
    n, d = data.shape
    e = edge_index_pos.shape[1]
    fn = _build(e, d)

    def prep(ei):
        return ei.astype(jnp.int32).reshape(-1)

    return fn(data, prep(edge_index_pos), prep(edge_index_neg))


# speedup vs baseline: 1.0154x; 1.0154x over previous
import functools

import jax
import jax.numpy as jnp
from jax import lax
from jax.experimental import pallas as pl
from jax.experimental.pallas import tpu as pltpu
from jax.experimental.pallas import tpu_sc as plsc

_EPC = 64
_NBUF = 3
_CPW = (53, 45)


@functools.lru_cache(maxsize=None)
def _build(e, d):
    info = plsc.get_sparse_core_info()
    nc = info.num_cores
    ns = info.num_subcores
    nw = nc * ns
    n_full = e // _EPC
    tail = e - n_full * _EPC
    n_chunks = n_full + (1 if tail else 0)
    cpw0, cpw1 = _CPW
    if nc != 2 or ns * (cpw0 + cpw1) < n_chunks:
        cpw0 = cpw1 = -(-n_chunks // nw)
    cpw_max = max(cpw0, cpw1)
    epw = cpw_max * _EPC
    last_base_e = (ns * cpw0 + (ns - 1) * cpw1) * _EPC
    last_sz = e - last_base_e
    zfill = (epw - last_sz) // 16
    rounds = -(-cpw_max // _NBUF) + 1

    mesh = plsc.VectorSubcoreMesh(core_axis_name="c", subcore_axis_name="s")

    @functools.partial(
        pl.kernel,
        mesh=mesh,
        out_type=(
            jax.ShapeDtypeStruct((e, 2 * d), jnp.float32),
            jax.ShapeDtypeStruct((e, 2 * d), jnp.float32),
        ),
        scratch_types=(
            [pltpu.VMEM((epw,), jnp.int32) for _ in range(4)]
            + [pltpu.VMEM((_EPC, 2 * d), jnp.float32) for _ in range(2 * _NBUF)]
            + [pltpu.SemaphoreType.DMA for _ in range(4 * _NBUF)]
        ),
    )
    def gather_kernel(data_hbm, eip_hbm, ein_hbm, outp_hbm, outn_hbm, *sc):
        ed = [[sc[2 * s + h] for h in range(2)] for s in range(2)]
        rows = [[sc[4 + s * _NBUF + b] for b in range(_NBUF)] for s in range(2)]
        o = 4 + 2 * _NBUF
        gsem = [[sc[o + s * _NBUF + b] for b in range(_NBUF)] for s in range(2)]
        o += 2 * _NBUF
        wsem = [[sc[o + s * _NBUF + b] for b in range(_NBUF)] for s in range(2)]
        outs = (outp_hbm, outn_hbm)
        eis = (eip_hbm, ein_hbm)

        c_idx = lax.axis_index("c")
        s_idx = lax.axis_index("s")
        base_chunk = jnp.where(c_idx == 0, s_idx * cpw0,
                               ns * cpw0 + s_idx * cpw1)
        my_cpw = jnp.where(c_idx == 0, cpw0, cpw1)
        base_e = base_chunk * _EPC
        is_last = jnp.logical_and(c_idx == nc - 1, s_idx == ns - 1)

        @pl.when(jnp.logical_not(is_last))
        def _stage_full():
            for s in range(2):
                for h in range(2):
                    pltpu.async_copy(eis[s].at[pl.ds(h * e + base_e, epw)],
                                     ed[s][h], gsem[s][h])
            for s in range(2):
                for h in range(2):
                    pltpu.make_async_copy(eis[s].at[pl.ds(h * e + base_e, epw)],
                                          ed[s][h], gsem[s][h]).wait()

        @pl.when(is_last)
        def _stage_last():
            for s in range(2):
                pltpu.async_copy(eis[s].at[pl.ds(base_e, epw)],
                                 ed[s][0], gsem[s][0])
                pltpu.async_copy(eis[s].at[pl.ds(e + base_e, last_sz)],
                                 ed[s][1].at[pl.ds(0, last_sz)], gsem[s][1])
            for s in range(2):
                pltpu.make_async_copy(eis[s].at[pl.ds(base_e, epw)],
                                      ed[s][0], gsem[s][0]).wait()
                pltpu.make_async_copy(eis[s].at[pl.ds(e + base_e, last_sz)],
                                      ed[s][1].at[pl.ds(0, last_sz)],
                                      gsem[s][1]).wait()
            zeros = jnp.zeros((16,), jnp.int32)
            for s in range(2):
                for g in range(zfill):
                    ed[s][1][pl.ds(last_sz + g * 16, 16)] = zeros

        def write_desc(s, b, cg, wait):
            @pl.when(cg < n_full)
            def _full():
                cp = pltpu.make_async_copy(
                    rows[s][b], outs[s].at[pl.ds(cg * _EPC, _EPC)], wsem[s][b])
                cp.wait() if wait else cp.start()
            if tail:
                @pl.when(cg == n_full)
                def _part():
                    cp = pltpu.make_async_copy(
                        rows[s][b].at[pl.ds(0, tail)],
                        outs[s].at[pl.ds(n_full * _EPC, tail)], wsem[s][b])
                    cp.wait() if wait else cp.start()

        def gather_desc(s, b, c_rel, wait):
            for h in range(2):
                cp = pltpu.make_async_copy(
                    data_hbm.at[ed[s][h].at[pl.ds(c_rel * _EPC, _EPC)]],
                    rows[s][b].at[:, pl.ds(h * d, d)], gsem[s][b])
                cp.wait() if wait else cp.start()

        def round_(j, carry):
            for b in range(_NBUF):
                c_new = j * _NBUF + b
                c_mid = c_new - _NBUF

                @pl.when(jnp.logical_and(c_mid >= 0, c_mid < my_cpw))
                def _drain_and_write():
                    for s in (0, 1):
                        gather_desc(s, b, c_mid, wait=True)
                    for s in (0, 1):
                        write_desc(s, b, base_chunk + c_mid, wait=False)

                @pl.when(c_new < my_cpw)
                def _fire():
                    @pl.when(c_mid >= 0)
                    def _wait_prev_write():
                        for s in (0, 1):
                            write_desc(s, b, base_chunk + c_mid, wait=True)
                    for s in (0, 1):
                        gather_desc(s, b, c_new, wait=False)

            return carry

        lax.fori_loop(0, rounds, round_, 0)

        for b in range(_NBUF):
            last_c = ((my_cpw - 1 - b) // _NBUF) * _NBUF + b

            @pl.when(last_c >= 0)
            def _drain_final():
                for s in (0, 1):
                    write_desc(s, b, base_chunk + last_c, wait=True)

    return gather_kernel


def kernel(data, edge_index_pos, edge_index_neg):
    n, d = data.shape
    e = edge_index_pos.shape[1]
    fn = _build(e, d)

    def prep(ei):
        return ei.astype(jnp.int32).reshape(-1)

    return fn(data, prep(edge_index_pos), prep(edge_index_neg))


# speedup vs baseline: 1.0163x; 1.0009x over previous
import functools

import jax
import jax.numpy as jnp
from jax import lax
from jax.experimental import pallas as pl
from jax.experimental.pallas import tpu as pltpu
from jax.experimental.pallas import tpu_sc as plsc

_EPC = 64
_NBUF = 3
_CPW = (49, 49)


@functools.lru_cache(maxsize=None)
def _build(e, d):
    info = plsc.get_sparse_core_info()
    nc = info.num_cores
    ns = info.num_subcores
    nw = nc * ns
    n_full = e // _EPC
    tail = e - n_full * _EPC
    n_chunks = n_full + (1 if tail else 0)
    cpw0, cpw1 = _CPW
    if nc != 2 or ns * (cpw0 + cpw1) < n_chunks:
        cpw0 = cpw1 = -(-n_chunks // nw)
    cpw_max = max(cpw0, cpw1)
    epw = cpw_max * _EPC
    last_base_e = (ns * cpw0 + (ns - 1) * cpw1) * _EPC
    last_sz = e - last_base_e
    zfill = (epw - last_sz) // 16
    rounds = -(-cpw_max // _NBUF) + 1

    mesh = plsc.VectorSubcoreMesh(core_axis_name="c", subcore_axis_name="s")

    @functools.partial(
        pl.kernel,
        mesh=mesh,
        out_type=(
            jax.ShapeDtypeStruct((e, 2 * d), jnp.float32),
            jax.ShapeDtypeStruct((e, 2 * d), jnp.float32),
        ),
        scratch_types=(
            [pltpu.VMEM((epw,), jnp.int32) for _ in range(4)]
            + [pltpu.VMEM((_EPC, 2 * d), jnp.float32) for _ in range(2 * _NBUF)]
            + [pltpu.SemaphoreType.DMA for _ in range(4 * _NBUF)]
        ),
    )
    def gather_kernel(data_hbm, eip_hbm, ein_hbm, outp_hbm, outn_hbm, *sc):
        ed = [[sc[2 * s + h] for h in range(2)] for s in range(2)]
        rows = [[sc[4 + s * _NBUF + b] for b in range(_NBUF)] for s in range(2)]
        o = 4 + 2 * _NBUF
        gsem = [[sc[o + s * _NBUF + b] for b in range(_NBUF)] for s in range(2)]
        o += 2 * _NBUF
        wsem = [[sc[o + s * _NBUF + b] for b in range(_NBUF)] for s in range(2)]
        outs = (outp_hbm, outn_hbm)
        eis = (eip_hbm, ein_hbm)

        c_idx = lax.axis_index("c")
        s_idx = lax.axis_index("s")
        base_chunk = jnp.where(c_idx == 0, s_idx * cpw0,
                               ns * cpw0 + s_idx * cpw1)
        my_cpw = jnp.where(c_idx == 0, cpw0, cpw1)
        base_e = base_chunk * _EPC
        is_last = jnp.logical_and(c_idx == nc - 1, s_idx == ns - 1)

        @pl.when(jnp.logical_not(is_last))
        def _stage_full():
            for s in range(2):
                for h in range(2):
                    pltpu.async_copy(eis[s].at[pl.ds(h * e + base_e, epw)],
                                     ed[s][h], gsem[s][h])
            for s in range(2):
                for h in range(2):
                    pltpu.make_async_copy(eis[s].at[pl.ds(h * e + base_e, epw)],
                                          ed[s][h], gsem[s][h]).wait()

        @pl.when(is_last)
        def _stage_last():
            for s in range(2):
                pltpu.async_copy(eis[s].at[pl.ds(base_e, epw)],
                                 ed[s][0], gsem[s][0])
                pltpu.async_copy(eis[s].at[pl.ds(e + base_e, last_sz)],
                                 ed[s][1].at[pl.ds(0, last_sz)], gsem[s][1])
            for s in range(2):
                pltpu.make_async_copy(eis[s].at[pl.ds(base_e, epw)],
                                      ed[s][0], gsem[s][0]).wait()
                pltpu.make_async_copy(eis[s].at[pl.ds(e + base_e, last_sz)],
                                      ed[s][1].at[pl.ds(0, last_sz)],
                                      gsem[s][1]).wait()
            zeros = jnp.zeros((16,), jnp.int32)
            for s in range(2):
                for g in range(zfill):
                    ed[s][1][pl.ds(last_sz + g * 16, 16)] = zeros

        def write_desc(s, b, cg, wait):
            @pl.when(cg < n_full)
            def _full():
                cp = pltpu.make_async_copy(
                    rows[s][b], outs[s].at[pl.ds(cg * _EPC, _EPC)], wsem[s][b])
                cp.wait() if wait else cp.start()
            if tail:
                @pl.when(cg == n_full)
                def _part():
                    cp = pltpu.make_async_copy(
                        rows[s][b].at[pl.ds(0, tail)],
                        outs[s].at[pl.ds(n_full * _EPC, tail)], wsem[s][b])
                    cp.wait() if wait else cp.start()

        def gather_desc(s, b, c_rel, wait):
            for h in range(2):
                cp = pltpu.make_async_copy(
                    data_hbm.at[ed[s][h].at[pl.ds(c_rel * _EPC, _EPC)]],
                    rows[s][b].at[:, pl.ds(h * d, d)], gsem[s][b])
                cp.wait() if wait else cp.start()

        def round_(j, carry):
            for b in range(_NBUF):
                c_new = j * _NBUF + b
                c_mid = c_new - _NBUF

                @pl.when(jnp.logical_and(c_mid >= 0, c_mid < my_cpw))
                def _drain_and_write():
                    for s in (0, 1):
                        gather_desc(s, b, c_mid, wait=True)
                    for s in (0, 1):
                        write_desc(s, b, base_chunk + c_mid, wait=False)

                @pl.when(c_new < my_cpw)
                def _fire():
                    @pl.when(c_mid >= 0)
                    def _wait_prev_write():
                        for s in (0, 1):
                            write_desc(s, b, base_chunk + c_mid, wait=True)
                    for s in (0, 1):
                        gather_desc(s, b, c_new, wait=False)

            return carry

        lax.fori_loop(0, rounds, round_, 0)

        for b in range(_NBUF):
            last_c = ((my_cpw - 1 - b) // _NBUF) * _NBUF + b

            @pl.when(last_c >= 0)
            def _drain_final():
                for s in (0, 1):
                    write_desc(s, b, base_chunk + last_c, wait=True)

    return gather_kernel


def kernel(data, edge_index_pos, edge_index_neg):
    n, d = data.shape
    e = edge_index_pos.shape[1]
    fn = _build(e, d)

    def prep(ei):
        return ei.astype(jnp.int32).reshape(-1)

    return fn(data, prep(edge_index_pos), prep(edge_index_neg))
